# baseline (speedup 1.0000x reference)
import functools

import jax
import jax.numpy as jnp
import numpy as np
from jax.experimental import pallas as pl
from jax.experimental.pallas import tpu as pltpu

EPS = 1e-6
KNN = 20
ZC = 170
THETA_C = 64
PB = 128
N = 1024
E = KNN * PB


def _split3(a):
    h = a.astype(jnp.bfloat16)
    r = a - h.astype(jnp.float32)
    l = r.astype(jnp.bfloat16)
    l2 = (r - l.astype(jnp.float32)).astype(jnp.bfloat16)
    return h, l, l2


def _edge_body(ft_ref, ftT_ref, wfe_ref, wde_ref, oft_ref, oftT_ref, oh_ref,
               nbr_ref, *, cross, slope, cp, op):
    j = pl.program_id(1)
    NB = N // PB
    ft = ft_ref[0]
    ftT = ftT_ref[0]

    ftb_cur = ft_ref[0, pl.ds(jnp.minimum(j, NB - 1) * PB, PB), :]
    G = jnp.dot(ftb_cur, ftT, preferred_element_type=jnp.float32)
    xxall = jnp.sum(ftT * ftT, axis=0, keepdims=True)
    xxb = jnp.sum(ftb_cur * ftb_cur, axis=1, keepdims=True)
    inner = -2.0 * G
    D = -xxb - inner - xxall
    iota = jax.lax.broadcasted_iota(jnp.int32, (PB, N), 1)
    par = j % 2
    fh, fl, fl2 = _split3(ft)

    def step(k, prev):
        Dm = jnp.where(D < prev, D, -jnp.inf)
        mk = jnp.max(Dm, axis=1, keepdims=True)
        cand = jnp.where(Dm == mk, iota, jnp.int32(1 << 30))
        ik = jnp.min(cand, axis=1, keepdims=True)
        oh_ref[par, pl.ds(k * PB, PB), :] = (cand == ik).astype(jnp.bfloat16)
        OHk = oh_ref[1 - par, pl.ds(k * PB, PB), :]
        nbr_ref[pl.ds(k * PB, PB), :] = (
            jnp.dot(OHk, fh, preferred_element_type=jnp.float32)
            + jnp.dot(OHk, fl, preferred_element_type=jnp.float32)
            + jnp.dot(OHk, fl2, preferred_element_type=jnp.float32))
        return mk

    jax.lax.fori_loop(0, KNN, step,
                      jnp.full((PB, 1), jnp.inf, jnp.float32))

    @pl.when(j > 0)
    def _conv():
        jp = jnp.maximum(j - 1, 0)
        ftb = ft_ref[0, pl.ds(jp * PB, PB), :]
        nbr = nbr_ref[...]
        ctr = jnp.concatenate([ftb] * KNN, axis=0)

        nb = [nbr[:, d * cp:(d + 1) * cp] for d in range(3)]
        cb = [ctr[:, d * cp:(d + 1) * cp] for d in range(3)]
        ps, ds_ = [], []
        for d in range(3):
            if cross:
                a, b = (d + 1) % 3, (d + 2) % 3
                cr = nb[a] * cb[b] - nb[b] * cb[a]
                e = jnp.concatenate([nb[d] - cb[d], cb[d], cr], axis=1)
            else:
                e = jnp.concatenate([nb[d] - cb[d], cb[d]], axis=1)
            ps.append(jnp.dot(e, wfe_ref[...], preferred_element_type=jnp.float32))
            ds_.append(jnp.dot(e, wde_ref[...], preferred_element_type=jnp.float32))

        dot = ps[0] * ds_[0] + ps[1] * ds_[1] + ps[2] * ds_[2]
        dsq = ds_[0] * ds_[0] + ds_[1] * ds_[1] + ds_[2] * ds_[2]
        neg = dot < 0
        scale = dot / (dsq + EPS)
        outs = []
        for d in range(3):
            o = slope * ps[d] + (1.0 - slope) * jnp.where(neg, ps[d] - scale * ds_[d], ps[d])
            acc = o[0:PB]
            for k in range(1, KNN):
                acc = acc + o[k * PB:(k + 1) * PB]
            outs.append(acc * (1.0 / KNN))
        feat = jnp.concatenate(outs, axis=1)
        oft_ref[0] = feat
        oftT_ref[0] = jnp.transpose(feat)


def _edge_layer(ft, ftT, wfe, wde, cross, slope, op):
    B = ft.shape[0]
    L = ft.shape[2]
    cp = L // 3
    Ce = wfe.shape[0]
    body = functools.partial(_edge_body, cross=cross, slope=slope, cp=cp, op=op)
    NB = N // PB
    oft, oftT = pl.pallas_call(
        body,
        grid=(B, NB + 1),
        in_specs=[
            pl.BlockSpec((1, N, L), lambda b, j: (b, 0, 0)),
            pl.BlockSpec((1, L, N), lambda b, j: (b, 0, 0)),
            pl.BlockSpec((Ce, op), lambda b, j: (0, 0)),
            pl.BlockSpec((Ce, op), lambda b, j: (0, 0)),
        ],
        out_specs=[
            pl.BlockSpec((1, PB, 3 * op),
                         lambda b, j: (b, jnp.maximum(j - 1, 0), 0)),
            pl.BlockSpec((1, 3 * op, PB),
                         lambda b, j: (b, 0, jnp.maximum(j - 1, 0))),
        ],
        out_shape=[
            jax.ShapeDtypeStruct((B, N, 3 * op), jnp.float32),
            jax.ShapeDtypeStruct((B, 3 * op, N), jnp.float32),
        ],
        scratch_shapes=[pltpu.VMEM((2, E, N), jnp.bfloat16),
                        pltpu.VMEM((E, 3 * cp), jnp.float32)],
    )(ft, ftT, wfe, wde)
    return oft, oftT


def _pack_w(W, cp, op):
    O, C2 = W.shape
    C = C2 // 2
    We = jnp.zeros((2 * cp, op), jnp.float32)
    We = We.at[:C, :O].set(W[:, :C].T)
    We = We.at[cp:cp + C, :O].set(W[:, C:].T)
    return We


def _tail_kernel(f1_ref, f2_ref, f3_ref, f4_ref,
                 wf5_ref, wd5_ref, w1f_ref, w1d_ref, w2f_ref, w2d_ref,
                 wsl_ref, wt1_ref, wt2_ref, wt3_ref, b3_ref,
                 out_ref, z_ref, t2_ref):
    j = pl.program_id(1)

    @pl.when(j == 0)
    def _():
        def vn(ps, ds_, slope):
            dot = ps[0] * ds_[0] + ps[1] * ds_[1] + ps[2] * ds_[2]
            dsq = ds_[0] * ds_[0] + ds_[1] * ds_[1] + ds_[2] * ds_[2]
            neg = dot < 0
            sc = dot / (dsq + EPS)
            return [slope * p + (1.0 - slope) * jnp.where(neg, p - sc * d, p)
                    for p, d in zip(ps, ds_)]

        xcs, p5s, d5s = [], [], []
        for d in range(3):
            xc = jnp.concatenate([
                f1_ref[0, d * 32:(d + 1) * 32, :],
                f2_ref[0, d * 32:(d + 1) * 32, :],
                f3_ref[0, d * 64:(d + 1) * 64, :],
                f4_ref[0, d * 128:(d + 1) * 128, :]], axis=0)
            xcs.append(xc)
            p5s.append(jnp.dot(wf5_ref[...], xc, preferred_element_type=jnp.float32))
            d5s.append(jnp.dot(wd5_ref[...], xc,
                               preferred_element_type=jnp.float32)[0:1])
        h5s = vn(p5s, d5s, 0.2)
        zs = [jnp.mean(h, axis=1, keepdims=True) for h in h5s]
        xx2s = [jnp.concatenate([h, jnp.broadcast_to(z, h.shape)], axis=0)
                for h, z in zip(h5s, zs)]
        p1s = [jnp.dot(w1f_ref[...], xx, preferred_element_type=jnp.float32)
               for xx in xx2s]
        d1s = [jnp.dot(w1d_ref[...], xx, preferred_element_type=jnp.float32)
               for xx in xx2s]
        z1s = vn(p1s, d1s, 0.2)
        p2s = [jnp.dot(w2f_ref[...], z1, preferred_element_type=jnp.float32)
               for z1 in z1s]
        d2s = [jnp.dot(w2d_ref[...], z1, preferred_element_type=jnp.float32)
               for z1 in z1s]
        z2s = vn(p2s, d2s, 0.2)
        z0s = [jnp.dot(wsl_ref[...], z2, preferred_element_type=jnp.float32)
               for z2 in z2s]
        xrs = []
        for k in range(3):
            xs = (xx2s[0] * z0s[k][0:1, :]
                  + xx2s[1] * z0s[k][1:2, :]
                  + xx2s[2] * z0s[k][2:3, :])
            xrs.append(xs)
        xr = jnp.concatenate(xrs, axis=0)
        t1 = jax.nn.relu(jnp.dot(wt1_ref[...], xr,
                                 preferred_element_type=jnp.float32))
        t2_ref[...] = jax.nn.relu(jnp.dot(wt2_ref[...], t1,
                                          preferred_element_type=jnp.float32))
        z_ref[0] = jnp.concatenate(
            [zs[0], zs[1], zs[2], jnp.zeros((176, 125), jnp.float32)], axis=1)

    out_ref[0] = (jnp.dot(wt3_ref[...], t2_ref[...],
                          preferred_element_type=jnp.float32)
                  + b3_ref[0, 0].reshape(-1, 1))


def _pack_tail_weights(Wf5, Wd5, Ws1f, Ws1d, Ws2f, Ws2d, Wsl, Wt1):
    def pack5(W):
        rows = W.shape[0]
        Wp = jnp.zeros((176 if rows > 1 else 8, 256), jnp.float32)
        Wp = Wp.at[:rows, 0:21].set(W[:, 0:21])
        Wp = Wp.at[:rows, 32:53].set(W[:, 21:42])
        Wp = Wp.at[:rows, 64:106].set(W[:, 42:84])
        Wp = Wp.at[:rows, 128:213].set(W[:, 84:169])
        return Wp
    def pack1(W):
        Wp = jnp.zeros((176, 352), jnp.float32)
        Wp = Wp.at[:170, 0:170].set(W[:, 0:170])
        Wp = Wp.at[:170, 176:346].set(W[:, 170:340])
        return Wp
    w2f = jnp.zeros((88, 176), jnp.float32).at[:85, :170].set(Ws2f)
    w2d = jnp.zeros((88, 176), jnp.float32).at[:85, :170].set(Ws2d)
    wsl = jnp.zeros((8, 88), jnp.float32).at[:3, :85].set(Wsl)
    i = np.arange(340)
    ir = np.where(i < 170, i, 176 + i - 170)
    newcols = (np.arange(3)[:, None] * 352 + ir[None, :]).reshape(-1)
    origcols = (i[None, :] * 3 + np.arange(3)[:, None]).reshape(-1)
    wt1 = jnp.zeros((1020, 1056), jnp.float32).at[:, newcols].set(Wt1[:, origcols])
    return (pack5(Wf5), pack5(Wd5), pack1(Ws1f), pack1(Ws1d), w2f, w2d, wsl, wt1)


def _tail_theta(f1T, f2T, f3T, f4T, packed, Wt2, Wt3, bt3):
    B = f1T.shape[0]
    wf5, wd5, w1f, w1d, w2f, w2d, wsl, wt1 = packed
    M = Wt3.shape[0]
    MB = 640
    out, zout = pl.pallas_call(
        _tail_kernel,
        grid=(B, M // MB),
        in_specs=[
            pl.BlockSpec((1, 96, N), lambda b, j: (b, 0, 0)),
            pl.BlockSpec((1, 96, N), lambda b, j: (b, 0, 0)),
            pl.BlockSpec((1, 192, N), lambda b, j: (b, 0, 0)),
            pl.BlockSpec((1, 384, N), lambda b, j: (b, 0, 0)),
            pl.BlockSpec((176, 256), lambda b, j: (0, 0)),
            pl.BlockSpec((8, 256), lambda b, j: (0, 0)),
            pl.BlockSpec((176, 352), lambda b, j: (0, 0)),
            pl.BlockSpec((176, 352), lambda b, j: (0, 0)),
            pl.BlockSpec((88, 176), lambda b, j: (0, 0)),
            pl.BlockSpec((88, 176), lambda b, j: (0, 0)),
            pl.BlockSpec((8, 88), lambda b, j: (0, 0)),
            pl.BlockSpec((1020, 1056), lambda b, j: (0, 0)),
            pl.BlockSpec((1020, 1020), lambda b, j: (0, 0)),
            pl.BlockSpec((MB, 1020), lambda b, j: (j, 0)),
            pl.BlockSpec((1, 1, MB), lambda b, j: (j, 0, 0)),
        ],
        out_specs=[
            pl.BlockSpec((1, MB, N), lambda b, j: (b, j, 0)),
            pl.BlockSpec((1, 176, 128), lambda b, j: (b, 0, 0)),
        ],
        out_shape=[
            jax.ShapeDtypeStruct((B, M, N), jnp.float32),
            jax.ShapeDtypeStruct((B, 176, 128), jnp.float32),
        ],
        scratch_shapes=[pltpu.VMEM((1020, N), jnp.float32)],
    )(f1T, f2T, f3T, f4T, wf5, wd5, w1f, w1d, w2f, w2d, wsl,
      wt1, Wt2, Wt3, bt3.reshape(M // MB, 1, MB))
    return out, zout


def kernel(x, Wf1, Wd1, Wf2, Wd2, Wf3, Wd3, Wf4, Wd4, Wf5, Wd5,
           Ws1f, Ws1d, Ws2f, Ws2d, Wsl, Wt1, Wt2, Wt3, bt3):
    B = x.shape[0]

    ft0 = jnp.zeros((B, N, 24), jnp.float32).at[:, :, (0, 8, 16)].set(x)
    ftT0 = jnp.transpose(ft0, (0, 2, 1))
    def pack1(W):
        We = jnp.zeros((24, 32), jnp.float32)
        We = We.at[0, :21].set(W[:, 0])
        We = We.at[8, :21].set(W[:, 1])
        We = We.at[16, :21].set(W[:, 2])
        return We
    ft1, ft1T = _edge_layer(ft0, ftT0, pack1(Wf1), pack1(Wd1), True, 0.0, 32)

    ft2, ft2T = _edge_layer(ft1, ft1T, _pack_w(Wf2, 32, 32),
                            _pack_w(Wd2, 32, 32), False, 0.0, 32)
    ft3, ft3T = _edge_layer(ft2, ft2T, _pack_w(Wf3, 32, 64),
                            _pack_w(Wd3, 32, 64), False, 0.0, 64)
    _, ft4T = _edge_layer(ft3, ft3T, _pack_w(Wf4, 64, 128),
                          _pack_w(Wd4, 64, 128), False, 0.0, 128)

    packed = _pack_tail_weights(Wf5, Wd5, Ws1f, Ws1d, Ws2f, Ws2d, Wsl, Wt1)
    t3, zout = _tail_theta(ft1T, ft2T, ft3T, ft4T, packed, Wt2, Wt3, bt3)
    z = zout[:, :170, :3]
    theta = t3.reshape(B, THETA_C, ZC, N)
    return (z, theta)


# speedup vs baseline: 1.4738x; 1.4738x over previous
import functools

import jax
import jax.numpy as jnp
import numpy as np
from jax.experimental import pallas as pl
from jax.experimental.pallas import tpu as pltpu

EPS = 1e-6
KNN = 20
ZC = 170
THETA_C = 64
PB = 128
N = 1024
E = KNN * PB


def _split3(a):
    h = a.astype(jnp.bfloat16)
    r = a - h.astype(jnp.float32)
    l = r.astype(jnp.bfloat16)
    l2 = (r - l.astype(jnp.float32)).astype(jnp.bfloat16)
    return h, l, l2


def _edge_body(ft_ref, ftT_ref, wfe_ref, wde_ref, oft_ref, oftT_ref, oh_ref,
               *, cross, slope, cp, op):
    j = pl.program_id(1)
    ft = ft_ref[0]
    ftbT = ftT_ref[0, :, pl.ds(j * PB, PB)]

    G = jnp.dot(ft, ftbT, preferred_element_type=jnp.float32)
    xxm = jnp.sum(ft * ft, axis=1, keepdims=True)
    xxp = jnp.sum(ftbT * ftbT, axis=0, keepdims=True)
    inner = -2.0 * G
    D = -xxp - inner - xxm
    iota = jax.lax.broadcasted_iota(jnp.int32, (N, PB), 0)

    def step(k, prev):
        Dm = jnp.where(D < prev, D, -jnp.inf)
        mk = jnp.max(Dm, axis=0, keepdims=True)
        cand = jnp.where(Dm == mk, iota, jnp.int32(1 << 30))
        ik = jnp.min(cand, axis=0, keepdims=True)
        oh_ref[:, pl.ds(k * PB, PB)] = (cand == ik).astype(jnp.bfloat16)
        return mk

    jax.lax.fori_loop(0, KNN, step, jnp.full((1, PB), jnp.inf, jnp.float32))

    fh, fl, fl2 = _split3(ftT_ref[0])
    OHT = oh_ref[...]
    nbrT = (jnp.dot(fh, OHT, preferred_element_type=jnp.float32)
            + jnp.dot(fl, OHT, preferred_element_type=jnp.float32)
            + jnp.dot(fl2, OHT, preferred_element_type=jnp.float32))
    ctrT = jnp.concatenate([ftbT] * KNN, axis=1)

    nb = [nbrT[d * cp:(d + 1) * cp, :] for d in range(3)]
    cb = [ctrT[d * cp:(d + 1) * cp, :] for d in range(3)]
    ps, ds_ = [], []
    for d in range(3):
        if cross:
            a, b = (d + 1) % 3, (d + 2) % 3
            cr = nb[a] * cb[b] - nb[b] * cb[a]
            e = jnp.concatenate([nb[d] - cb[d], cb[d], cr], axis=0)
        else:
            e = jnp.concatenate([nb[d] - cb[d], cb[d]], axis=0)
        ps.append(jnp.dot(wfe_ref[...], e, preferred_element_type=jnp.float32))
        ds_.append(jnp.dot(wde_ref[...], e, preferred_element_type=jnp.float32))

    dot = ps[0] * ds_[0] + ps[1] * ds_[1] + ps[2] * ds_[2]
    dsq = ds_[0] * ds_[0] + ds_[1] * ds_[1] + ds_[2] * ds_[2]
    neg = dot < 0
    scale = dot / (dsq + EPS)
    outs = []
    for d in range(3):
        o = slope * ps[d] + (1.0 - slope) * jnp.where(neg, ps[d] - scale * ds_[d], ps[d])
        acc = o[:, 0:PB]
        for k in range(1, KNN):
            acc = acc + o[:, k * PB:(k + 1) * PB]
        outs.append(acc * (1.0 / KNN))
    feat = jnp.concatenate(outs, axis=0)
    oftT_ref[0] = feat
    oft_ref[0] = jnp.transpose(feat)


def _edge_layer(ft, ftT, wfe, wde, cross, slope, op):
    B = ft.shape[0]
    L = ft.shape[2]
    cp = L // 3
    Ce = wfe.shape[1]
    body = functools.partial(_edge_body, cross=cross, slope=slope, cp=cp, op=op)
    oft, oftT = pl.pallas_call(
        body,
        grid=(B, N // PB),
        in_specs=[
            pl.BlockSpec((1, N, L), lambda b, j: (b, 0, 0)),
            pl.BlockSpec((1, L, N), lambda b, j: (b, 0, 0)),
            pl.BlockSpec((op, Ce), lambda b, j: (0, 0)),
            pl.BlockSpec((op, Ce), lambda b, j: (0, 0)),
        ],
        out_specs=[
            pl.BlockSpec((1, PB, 3 * op), lambda b, j: (b, j, 0)),
            pl.BlockSpec((1, 3 * op, PB), lambda b, j: (b, 0, j)),
        ],
        out_shape=[
            jax.ShapeDtypeStruct((B, N, 3 * op), jnp.float32),
            jax.ShapeDtypeStruct((B, 3 * op, N), jnp.float32),
        ],
        scratch_shapes=[pltpu.VMEM((N, E), jnp.bfloat16)],
    )(ft, ftT, wfe, wde)
    return oft, oftT


def _pack_w(W, cp, op):
    O, C2 = W.shape
    C = C2 // 2
    We = jnp.zeros((op, 2 * cp), jnp.float32)
    We = We.at[:O, :C].set(W[:, :C])
    We = We.at[:O, cp:cp + C].set(W[:, C:])
    return We


def _tail_kernel(f1_ref, f2_ref, f3_ref, f4_ref,
                 wf5_ref, wd5_ref, w1f_ref, w1d_ref, w2f_ref, w2d_ref,
                 wsl_ref, wt1_ref, wt2_ref, wt3_ref, b3_ref,
                 out_ref, z_ref, t2_ref):
    j = pl.program_id(1)

    @pl.when(j == 0)
    def _():
        def vn(ps, ds_, slope):
            dot = ps[0] * ds_[0] + ps[1] * ds_[1] + ps[2] * ds_[2]
            dsq = ds_[0] * ds_[0] + ds_[1] * ds_[1] + ds_[2] * ds_[2]
            neg = dot < 0
            sc = dot / (dsq + EPS)
            return [slope * p + (1.0 - slope) * jnp.where(neg, p - sc * d, p)
                    for p, d in zip(ps, ds_)]

        xcs, p5s, d5s = [], [], []
        for d in range(3):
            xc = jnp.concatenate([
                f1_ref[0, d * 32:(d + 1) * 32, :],
                f2_ref[0, d * 32:(d + 1) * 32, :],
                f3_ref[0, d * 64:(d + 1) * 64, :],
                f4_ref[0, d * 128:(d + 1) * 128, :]], axis=0)
            xcs.append(xc)
            p5s.append(jnp.dot(wf5_ref[...], xc, preferred_element_type=jnp.float32))
            d5s.append(jnp.dot(wd5_ref[...], xc,
                               preferred_element_type=jnp.float32)[0:1])
        h5s = vn(p5s, d5s, 0.2)
        zs = [jnp.mean(h, axis=1, keepdims=True) for h in h5s]
        xx2s = [jnp.concatenate([h, jnp.broadcast_to(z, h.shape)], axis=0)
                for h, z in zip(h5s, zs)]
        p1s = [jnp.dot(w1f_ref[...], xx, preferred_element_type=jnp.float32)
               for xx in xx2s]
        d1s = [jnp.dot(w1d_ref[...], xx, preferred_element_type=jnp.float32)
               for xx in xx2s]
        z1s = vn(p1s, d1s, 0.2)
        p2s = [jnp.dot(w2f_ref[...], z1, preferred_element_type=jnp.float32)
               for z1 in z1s]
        d2s = [jnp.dot(w2d_ref[...], z1, preferred_element_type=jnp.float32)
               for z1 in z1s]
        z2s = vn(p2s, d2s, 0.2)
        z0s = [jnp.dot(wsl_ref[...], z2, preferred_element_type=jnp.float32)
               for z2 in z2s]
        xrs = []
        for k in range(3):
            xs = (xx2s[0] * z0s[k][0:1, :]
                  + xx2s[1] * z0s[k][1:2, :]
                  + xx2s[2] * z0s[k][2:3, :])
            xrs.append(xs)
        xr = jnp.concatenate(xrs, axis=0)
        t1 = jax.nn.relu(jnp.dot(wt1_ref[...], xr,
                                 preferred_element_type=jnp.float32))
        t2_ref[...] = jax.nn.relu(jnp.dot(wt2_ref[...], t1,
                                          preferred_element_type=jnp.float32))
        z_ref[0] = jnp.concatenate(
            [zs[0], zs[1], zs[2], jnp.zeros((176, 125), jnp.float32)], axis=1)

    out_ref[0] = (jnp.dot(wt3_ref[...], t2_ref[...],
                          preferred_element_type=jnp.float32)
                  + b3_ref[0, 0].reshape(-1, 1))


def _pack_tail_weights(Wf5, Wd5, Ws1f, Ws1d, Ws2f, Ws2d, Wsl, Wt1):
    def pack5(W):
        rows = W.shape[0]
        Wp = jnp.zeros((176 if rows > 1 else 8, 256), jnp.float32)
        Wp = Wp.at[:rows, 0:21].set(W[:, 0:21])
        Wp = Wp.at[:rows, 32:53].set(W[:, 21:42])
        Wp = Wp.at[:rows, 64:106].set(W[:, 42:84])
        Wp = Wp.at[:rows, 128:213].set(W[:, 84:169])
        return Wp
    def pack1(W):
        Wp = jnp.zeros((176, 352), jnp.float32)
        Wp = Wp.at[:170, 0:170].set(W[:, 0:170])
        Wp = Wp.at[:170, 176:346].set(W[:, 170:340])
        return Wp
    w2f = jnp.zeros((88, 176), jnp.float32).at[:85, :170].set(Ws2f)
    w2d = jnp.zeros((88, 176), jnp.float32).at[:85, :170].set(Ws2d)
    wsl = jnp.zeros((8, 88), jnp.float32).at[:3, :85].set(Wsl)
    i = np.arange(340)
    ir = np.where(i < 170, i, 176 + i - 170)
    newcols = (np.arange(3)[:, None] * 352 + ir[None, :]).reshape(-1)
    origcols = (i[None, :] * 3 + np.arange(3)[:, None]).reshape(-1)
    wt1 = jnp.zeros((1020, 1056), jnp.float32).at[:, newcols].set(Wt1[:, origcols])
    return (pack5(Wf5), pack5(Wd5), pack1(Ws1f), pack1(Ws1d), w2f, w2d, wsl, wt1)


def _tail_theta(f1T, f2T, f3T, f4T, packed, Wt2, Wt3, bt3):
    B = f1T.shape[0]
    wf5, wd5, w1f, w1d, w2f, w2d, wsl, wt1 = packed
    M = Wt3.shape[0]
    MB = 640
    out, zout = pl.pallas_call(
        _tail_kernel,
        grid=(B, M // MB),
        in_specs=[
            pl.BlockSpec((1, 96, N), lambda b, j: (b, 0, 0)),
            pl.BlockSpec((1, 96, N), lambda b, j: (b, 0, 0)),
            pl.BlockSpec((1, 192, N), lambda b, j: (b, 0, 0)),
            pl.BlockSpec((1, 384, N), lambda b, j: (b, 0, 0)),
            pl.BlockSpec((176, 256), lambda b, j: (0, 0)),
            pl.BlockSpec((8, 256), lambda b, j: (0, 0)),
            pl.BlockSpec((176, 352), lambda b, j: (0, 0)),
            pl.BlockSpec((176, 352), lambda b, j: (0, 0)),
            pl.BlockSpec((88, 176), lambda b, j: (0, 0)),
            pl.BlockSpec((88, 176), lambda b, j: (0, 0)),
            pl.BlockSpec((8, 88), lambda b, j: (0, 0)),
            pl.BlockSpec((1020, 1056), lambda b, j: (0, 0)),
            pl.BlockSpec((1020, 1020), lambda b, j: (0, 0)),
            pl.BlockSpec((MB, 1020), lambda b, j: (j, 0)),
            pl.BlockSpec((1, 1, MB), lambda b, j: (j, 0, 0)),
        ],
        out_specs=[
            pl.BlockSpec((1, MB, N), lambda b, j: (b, j, 0)),
            pl.BlockSpec((1, 176, 128), lambda b, j: (b, 0, 0)),
        ],
        out_shape=[
            jax.ShapeDtypeStruct((B, M, N), jnp.float32),
            jax.ShapeDtypeStruct((B, 176, 128), jnp.float32),
        ],
        scratch_shapes=[pltpu.VMEM((1020, N), jnp.float32)],
    )(f1T, f2T, f3T, f4T, wf5, wd5, w1f, w1d, w2f, w2d, wsl,
      wt1, Wt2, Wt3, bt3.reshape(M // MB, 1, MB))
    return out, zout


def kernel(x, Wf1, Wd1, Wf2, Wd2, Wf3, Wd3, Wf4, Wd4, Wf5, Wd5,
           Ws1f, Ws1d, Ws2f, Ws2d, Wsl, Wt1, Wt2, Wt3, bt3):
    B = x.shape[0]

    ft0 = jnp.zeros((B, N, 24), jnp.float32).at[:, :, (0, 8, 16)].set(x)
    ftT0 = jnp.transpose(ft0, (0, 2, 1))
    def pack1(W):
        We = jnp.zeros((32, 24), jnp.float32)
        We = We.at[:21, 0].set(W[:, 0])
        We = We.at[:21, 8].set(W[:, 1])
        We = We.at[:21, 16].set(W[:, 2])
        return We
    ft1, ft1T = _edge_layer(ft0, ftT0, pack1(Wf1), pack1(Wd1), True, 0.0, 32)

    ft2, ft2T = _edge_layer(ft1, ft1T, _pack_w(Wf2, 32, 32),
                            _pack_w(Wd2, 32, 32), False, 0.0, 32)
    ft3, ft3T = _edge_layer(ft2, ft2T, _pack_w(Wf3, 32, 64),
                            _pack_w(Wd3, 32, 64), False, 0.0, 64)
    _, ft4T = _edge_layer(ft3, ft3T, _pack_w(Wf4, 64, 128),
                          _pack_w(Wd4, 64, 128), False, 0.0, 128)

    packed = _pack_tail_weights(Wf5, Wd5, Ws1f, Ws1d, Ws2f, Ws2d, Wsl, Wt1)
    t3, zout = _tail_theta(ft1T, ft2T, ft3T, ft4T, packed, Wt2, Wt3, bt3)
    z = zout[:, :170, :3]
    theta = t3.reshape(B, THETA_C, ZC, N)
    return (z, theta)
